# baseline (speedup 1.0000x reference)
import functools

import jax
import jax.numpy as jnp
from jax import lax
from jax.experimental import pallas as pl
from jax.experimental.pallas import tpu as pltpu
from jax.experimental.pallas import tpu_sc as plsc

D = 256
K = 8192
N = 16 * 576

BN = 512
NN = N // BN
CH = 2048
NCH = K // CH

_INF = float("inf")


def _norms_body(x_ref, cb_ref, a2_ref, b2_ref):
    x = x_ref[...]
    cb = cb_ref[...]
    a2_ref[0, :] = jnp.sum(x * x, axis=1)
    b2_ref[:, 0] = jnp.sum(cb * cb, axis=1)


_norms_call = pl.pallas_call(
    _norms_body,
    out_shape=[
        jax.ShapeDtypeStruct((1, N), jnp.float32),
        jax.ShapeDtypeStruct((K, 1), jnp.float32),
    ],
)


def _preimage_threshold(m2):
    m2c = jnp.maximum(m2, 0.0)
    sk = jnp.sqrt(m2c)
    s_next = lax.bitcast_convert_type(
        lax.bitcast_convert_type(sk, jnp.int32) + 1, jnp.float32)
    pb = lax.bitcast_convert_type(sk * s_next, jnp.int32)
    t = m2c
    for db in (-2, -1, 0, 1, 2):
        c = lax.bitcast_convert_type(jnp.maximum(pb + db, 0), jnp.float32)
        t = jnp.where((jnp.sqrt(c) == sk) & (c > t), c, t)
    return t, sk


def _argmin_body(x_ref, cb_ref, a2_ref, b2_ref, out_ref, rowf_s):
    n = pl.program_id(0)

    @pl.when(n == 0)
    def _prologue():
        rowf_s[...] = lax.broadcasted_iota(jnp.int32, (CH, BN), 0).astype(
            jnp.float32)

    x2 = x_ref[...] * 2.0
    a2 = a2_ref[...]

    best_s = jnp.full((1, BN), _INF, dtype=jnp.float32)
    best_i = jnp.zeros((1, BN), dtype=jnp.float32)
    for j in range(NCH):
        cbj = cb_ref[pl.ds(j * CH, CH), :]
        s2 = lax.dot_general(cbj, x2, (((1,), (1,)), ((), ())),
                             preferred_element_type=jnp.float32)
        b2j = b2_ref[pl.ds(j * CH, CH), :]
        vj = a2 + b2j - s2
        mj = jnp.min(vj, axis=0, keepdims=True)
        tj, skj = _preimage_threshold(mj)
        lij = jnp.min(jnp.where(vj <= tj, rowf_s[...], _INF),
                      axis=0, keepdims=True)
        upd = skj < best_s
        best_i = jnp.where(upd, lij + jnp.float32(j * CH), best_i)
        best_s = jnp.where(upd, skj, best_s)

    out_ref[0, 0, :] = best_i.astype(jnp.int32)[0, :]


_argmin_call = pl.pallas_call(
    _argmin_body,
    grid=(NN,),
    in_specs=[
        pl.BlockSpec((BN, D), lambda n: (n, 0)),
        pl.BlockSpec((K, D), lambda n: (0, 0)),
        pl.BlockSpec((1, BN), lambda n: (0, n)),
        pl.BlockSpec((K, 1), lambda n: (0, 0)),
    ],
    out_specs=pl.BlockSpec((1, 1, BN), lambda n: (n, 0, 0)),
    out_shape=jax.ShapeDtypeStruct((NN, 1, BN), jnp.int32),
    scratch_shapes=[
        pltpu.VMEM((CH, BN), jnp.float32),
    ],
    compiler_params=pltpu.CompilerParams(
        dimension_semantics=("arbitrary",),
    ),
)


_NW = 32
_BPW = N // _NW


def _gather_body(table_hbm, idx_hbm, out_hbm, idx_v, rows_v, sem):
    wid = lax.axis_index("s") * 2 + lax.axis_index("c")
    base = wid * _BPW
    pltpu.sync_copy(idx_hbm.at[pl.ds(base, _BPW)], idx_v)
    pltpu.async_copy(table_hbm.at[idx_v], rows_v, sem).wait()
    pltpu.sync_copy(rows_v, out_hbm.at[pl.ds(base, _BPW)])


@functools.cache
def _gather_call():
    return functools.partial(
        pl.kernel,
        out_type=jax.ShapeDtypeStruct((N, D), jnp.float32),
        mesh=plsc.VectorSubcoreMesh(core_axis_name="c", subcore_axis_name="s"),
        scratch_types=[
            pltpu.VMEM((_BPW,), jnp.int32),
            pltpu.VMEM((_BPW, D), jnp.float32),
            pltpu.SemaphoreType.DMA,
        ],
    )(_gather_body)


def kernel(x, codebook):
    xf = x.reshape(N, D)
    a2, b2 = _norms_call(xf, codebook)
    idx_blocks = _argmin_call(xf, codebook, a2, b2)
    indices = idx_blocks.reshape(N)
    codes = _gather_call()(codebook, indices)
    idx_shape = list(x.shape)
    idx_shape[-1] = 1
    return codes.reshape(x.shape), indices.reshape(idx_shape)


# speedup vs baseline: 1.0076x; 1.0076x over previous
import functools

import jax
import jax.numpy as jnp
from jax import lax
from jax.experimental import pallas as pl
from jax.experimental.pallas import tpu as pltpu
from jax.experimental.pallas import tpu_sc as plsc

D = 256
K = 8192
N = 16 * 576

BN = 512
NN = N // BN
CH = 2048
NCH = K // CH

_INF = float("inf")


def _norms_body(x_ref, cb_ref, a2_ref, b2_ref):
    x = x_ref[...]
    cb = cb_ref[...]
    a2_ref[0, :] = jnp.sum(x * x, axis=1)
    b2_ref[:, 0] = jnp.sum(cb * cb, axis=1)


_norms_call = pl.pallas_call(
    _norms_body,
    out_shape=[
        jax.ShapeDtypeStruct((1, N), jnp.float32),
        jax.ShapeDtypeStruct((K, 1), jnp.float32),
    ],
)


def _preimage_threshold(m2):
    m2c = jnp.maximum(m2, 0.0)
    sk = jnp.sqrt(m2c)
    s_next = lax.bitcast_convert_type(
        lax.bitcast_convert_type(sk, jnp.int32) + 1, jnp.float32)
    pb = lax.bitcast_convert_type(sk * s_next, jnp.int32)
    t = jnp.zeros_like(m2c)
    for db in (-1, 0, 1):
        c = lax.bitcast_convert_type(jnp.maximum(pb + db, 0), jnp.float32)
        t = jnp.where(jnp.sqrt(c) == sk, c, t)
    return jnp.maximum(t, m2c), sk


def _argmin_body(x_ref, cb_ref, a2_ref, b2_ref, out_ref, rowf_s):
    n = pl.program_id(0)

    @pl.when(n == 0)
    def _prologue():
        rowf_s[...] = lax.broadcasted_iota(jnp.int32, (CH, BN), 0).astype(
            jnp.float32)

    x2 = x_ref[...] * 2.0
    a2 = a2_ref[...]

    best_s = jnp.full((BN,), _INF, dtype=jnp.float32)
    best_i = jnp.zeros((BN,), dtype=jnp.float32)
    for j in range(NCH):
        cbj = cb_ref[pl.ds(j * CH, CH), :]
        s2 = lax.dot_general(cbj, x2, (((1,), (1,)), ((), ())),
                             preferred_element_type=jnp.float32)
        b2j = b2_ref[pl.ds(j * CH, CH), :]
        vj = a2 + b2j - s2
        mj = jnp.min(vj, axis=0)
        tj, skj = _preimage_threshold(mj)
        lij = jnp.min(jnp.where(vj <= tj[None, :], rowf_s[...], _INF),
                      axis=0)
        upd = skj < best_s
        best_i = jnp.where(upd, lij + jnp.float32(j * CH), best_i)
        best_s = jnp.where(upd, skj, best_s)

    out_ref[0, 0, :] = best_i.astype(jnp.int32)


_argmin_call = pl.pallas_call(
    _argmin_body,
    grid=(NN,),
    in_specs=[
        pl.BlockSpec((BN, D), lambda n: (n, 0)),
        pl.BlockSpec((K, D), lambda n: (0, 0)),
        pl.BlockSpec((1, BN), lambda n: (0, n)),
        pl.BlockSpec((K, 1), lambda n: (0, 0)),
    ],
    out_specs=pl.BlockSpec((1, 1, BN), lambda n: (n, 0, 0)),
    out_shape=jax.ShapeDtypeStruct((NN, 1, BN), jnp.int32),
    scratch_shapes=[
        pltpu.VMEM((CH, BN), jnp.float32),
    ],
    compiler_params=pltpu.CompilerParams(
        dimension_semantics=("arbitrary",),
    ),
)


_NW = 32
_BPW = N // _NW


def _gather_body(table_hbm, idx_hbm, out_hbm, idx_v, rows_v, sem):
    wid = lax.axis_index("s") * 2 + lax.axis_index("c")
    base = wid * _BPW
    pltpu.sync_copy(idx_hbm.at[pl.ds(base, _BPW)], idx_v)
    pltpu.async_copy(table_hbm.at[idx_v], rows_v, sem).wait()
    pltpu.sync_copy(rows_v, out_hbm.at[pl.ds(base, _BPW)])


@functools.cache
def _gather_call():
    return functools.partial(
        pl.kernel,
        out_type=jax.ShapeDtypeStruct((N, D), jnp.float32),
        mesh=plsc.VectorSubcoreMesh(core_axis_name="c", subcore_axis_name="s"),
        scratch_types=[
            pltpu.VMEM((_BPW,), jnp.int32),
            pltpu.VMEM((_BPW, D), jnp.float32),
            pltpu.SemaphoreType.DMA,
        ],
    )(_gather_body)


def kernel(x, codebook):
    xf = x.reshape(N, D)
    a2, b2 = _norms_call(xf, codebook)
    idx_blocks = _argmin_call(xf, codebook, a2, b2)
    indices = idx_blocks.reshape(N)
    codes = _gather_call()(codebook, indices)
    idx_shape = list(x.shape)
    idx_shape[-1] = 1
    return codes.reshape(x.shape), indices.reshape(idx_shape)


# speedup vs baseline: 1.0481x; 1.0402x over previous
import functools

import jax
import jax.numpy as jnp
from jax import lax
from jax.experimental import pallas as pl
from jax.experimental.pallas import tpu as pltpu
from jax.experimental.pallas import tpu_sc as plsc

D = 256
K = 8192
N = 16 * 576

BN = 1024
NN = N // BN
CH = 2048
NCH = K // CH

_INF = float("inf")


def _norms_body(x_ref, cb_ref, a2_ref, b2_ref):
    x = x_ref[...]
    cb = cb_ref[...]
    a2_ref[0, :] = jnp.sum(x * x, axis=1)
    b2_ref[:, 0] = jnp.sum(cb * cb, axis=1)


_norms_call = pl.pallas_call(
    _norms_body,
    out_shape=[
        jax.ShapeDtypeStruct((1, N), jnp.float32),
        jax.ShapeDtypeStruct((K, 1), jnp.float32),
    ],
)


def _preimage_threshold(m2):
    m2c = jnp.maximum(m2, 0.0)
    sk = jnp.sqrt(m2c)
    s_next = lax.bitcast_convert_type(
        lax.bitcast_convert_type(sk, jnp.int32) + 1, jnp.float32)
    pb = lax.bitcast_convert_type(sk * s_next, jnp.int32)
    t = jnp.zeros_like(m2c)
    for db in (-1, 0, 1):
        c = lax.bitcast_convert_type(jnp.maximum(pb + db, 0), jnp.float32)
        t = jnp.where(jnp.sqrt(c) == sk, c, t)
    return jnp.maximum(t, m2c), sk


def _argmin_body(x_ref, cb_ref, a2_ref, b2_ref, out_ref, rowf_s):
    n = pl.program_id(0)

    @pl.when(n == 0)
    def _prologue():
        rowf_s[...] = lax.broadcasted_iota(jnp.int32, (CH, BN), 0).astype(
            jnp.float32)

    x2 = x_ref[...] * 2.0
    a2 = a2_ref[...]

    best_s = jnp.full((BN,), _INF, dtype=jnp.float32)
    best_i = jnp.zeros((BN,), dtype=jnp.float32)
    for j in range(NCH):
        cbj = cb_ref[pl.ds(j * CH, CH), :]
        s2 = lax.dot_general(cbj, x2, (((1,), (1,)), ((), ())),
                             preferred_element_type=jnp.float32)
        b2j = b2_ref[pl.ds(j * CH, CH), :]
        vj = a2 + b2j - s2
        mj = jnp.min(vj, axis=0)
        tj, skj = _preimage_threshold(mj)
        lij = jnp.min(jnp.where(vj <= tj[None, :], rowf_s[...], _INF),
                      axis=0)
        upd = skj < best_s
        best_i = jnp.where(upd, lij + jnp.float32(j * CH), best_i)
        best_s = jnp.where(upd, skj, best_s)

    out_ref[0, 0, :] = best_i.astype(jnp.int32)


_argmin_call = pl.pallas_call(
    _argmin_body,
    grid=(NN,),
    in_specs=[
        pl.BlockSpec((BN, D), lambda n: (n, 0)),
        pl.BlockSpec((K, D), lambda n: (0, 0)),
        pl.BlockSpec((1, BN), lambda n: (0, n)),
        pl.BlockSpec((K, 1), lambda n: (0, 0)),
    ],
    out_specs=pl.BlockSpec((1, 1, BN), lambda n: (n, 0, 0)),
    out_shape=jax.ShapeDtypeStruct((NN, 1, BN), jnp.int32),
    scratch_shapes=[
        pltpu.VMEM((CH, BN), jnp.float32),
    ],
    compiler_params=pltpu.CompilerParams(
        dimension_semantics=("arbitrary",),
    ),
)


_NW = 32
_BPW = N // _NW


def _gather_body(table_hbm, idx_hbm, out_hbm, idx_v, rows_v, sem):
    wid = lax.axis_index("s") * 2 + lax.axis_index("c")
    base = wid * _BPW
    pltpu.sync_copy(idx_hbm.at[pl.ds(base, _BPW)], idx_v)
    pltpu.async_copy(table_hbm.at[idx_v], rows_v, sem).wait()
    pltpu.sync_copy(rows_v, out_hbm.at[pl.ds(base, _BPW)])


@functools.cache
def _gather_call():
    return functools.partial(
        pl.kernel,
        out_type=jax.ShapeDtypeStruct((N, D), jnp.float32),
        mesh=plsc.VectorSubcoreMesh(core_axis_name="c", subcore_axis_name="s"),
        scratch_types=[
            pltpu.VMEM((_BPW,), jnp.int32),
            pltpu.VMEM((_BPW, D), jnp.float32),
            pltpu.SemaphoreType.DMA,
        ],
    )(_gather_body)


def kernel(x, codebook):
    xf = x.reshape(N, D)
    a2, b2 = _norms_call(xf, codebook)
    idx_blocks = _argmin_call(xf, codebook, a2, b2)
    indices = idx_blocks.reshape(N)
    codes = _gather_call()(codebook, indices)
    idx_shape = list(x.shape)
    idx_shape[-1] = 1
    return codes.reshape(x.shape), indices.reshape(idx_shape)


# speedup vs baseline: 1.0861x; 1.0363x over previous
import functools

import jax
import jax.numpy as jnp
from jax import lax
from jax.experimental import pallas as pl
from jax.experimental.pallas import tpu as pltpu
from jax.experimental.pallas import tpu_sc as plsc

D = 256
K = 8192
N = 16 * 576

BN = 1024
NN = N // BN
CH = 2048
NCH = K // CH

_INF = float("inf")


def _norms_body(x_ref, a2_ref):
    x = x_ref[...]
    a2_ref[0, :] = jnp.sum(x * x, axis=1)


_norms_call = pl.pallas_call(
    _norms_body,
    out_shape=jax.ShapeDtypeStruct((1, N), jnp.float32),
)


def _preimage_threshold(m2):
    m2c = jnp.maximum(m2, 0.0)
    sk = jnp.sqrt(m2c)
    s_next = lax.bitcast_convert_type(
        lax.bitcast_convert_type(sk, jnp.int32) + 1, jnp.float32)
    pb = lax.bitcast_convert_type(sk * s_next, jnp.int32)
    t = jnp.zeros_like(m2c)
    for db in (-1, 0, 1):
        c = lax.bitcast_convert_type(jnp.maximum(pb + db, 0), jnp.float32)
        t = jnp.where(jnp.sqrt(c) == sk, c, t)
    return jnp.maximum(t, m2c), sk


def _argmin_body(x_ref, cb_ref, a2_ref, out_ref, rowf_s, b2_s):
    n = pl.program_id(0)

    @pl.when(n == 0)
    def _prologue():
        rowf_s[...] = lax.broadcasted_iota(jnp.int32, (CH, BN), 0).astype(
            jnp.float32)
        cb = cb_ref[...]
        b2_s[:, 0] = jnp.sum(cb * cb, axis=1)

    x2 = x_ref[...] * 2.0
    a2 = a2_ref[...]

    best_s = jnp.full((BN,), _INF, dtype=jnp.float32)
    best_i = jnp.zeros((BN,), dtype=jnp.float32)
    for j in range(NCH):
        cbj = cb_ref[pl.ds(j * CH, CH), :]
        s2 = lax.dot_general(cbj, x2, (((1,), (1,)), ((), ())),
                             preferred_element_type=jnp.float32)
        b2j = b2_s[pl.ds(j * CH, CH), :]
        vj = a2 + b2j - s2
        mj = jnp.min(vj, axis=0)
        tj, skj = _preimage_threshold(mj)
        lij = jnp.min(jnp.where(vj <= tj[None, :], rowf_s[...], _INF),
                      axis=0)
        upd = skj < best_s
        best_i = jnp.where(upd, lij + jnp.float32(j * CH), best_i)
        best_s = jnp.where(upd, skj, best_s)

    out_ref[0, 0, :] = best_i.astype(jnp.int32)


_argmin_call = pl.pallas_call(
    _argmin_body,
    grid=(NN,),
    in_specs=[
        pl.BlockSpec((BN, D), lambda n: (n, 0)),
        pl.BlockSpec((K, D), lambda n: (0, 0)),
        pl.BlockSpec((1, BN), lambda n: (0, n)),
    ],
    out_specs=pl.BlockSpec((1, 1, BN), lambda n: (n, 0, 0)),
    out_shape=jax.ShapeDtypeStruct((NN, 1, BN), jnp.int32),
    scratch_shapes=[
        pltpu.VMEM((CH, BN), jnp.float32),
        pltpu.VMEM((K, 1), jnp.float32),
    ],
    compiler_params=pltpu.CompilerParams(
        dimension_semantics=("arbitrary",),
    ),
)


_NW = 32
_BPW = N // _NW


def _gather_body(table_hbm, idx_hbm, out_hbm, idx_v, rows_v, sem):
    wid = lax.axis_index("s") * 2 + lax.axis_index("c")
    base = wid * _BPW
    pltpu.sync_copy(idx_hbm.at[pl.ds(base, _BPW)], idx_v)
    pltpu.async_copy(table_hbm.at[idx_v], rows_v, sem).wait()
    pltpu.sync_copy(rows_v, out_hbm.at[pl.ds(base, _BPW)])


@functools.cache
def _gather_call():
    return functools.partial(
        pl.kernel,
        out_type=jax.ShapeDtypeStruct((N, D), jnp.float32),
        mesh=plsc.VectorSubcoreMesh(core_axis_name="c", subcore_axis_name="s"),
        scratch_types=[
            pltpu.VMEM((_BPW,), jnp.int32),
            pltpu.VMEM((_BPW, D), jnp.float32),
            pltpu.SemaphoreType.DMA,
        ],
    )(_gather_body)


def kernel(x, codebook):
    xf = x.reshape(N, D)
    a2 = _norms_call(xf)
    idx_blocks = _argmin_call(xf, codebook, a2)
    indices = idx_blocks.reshape(N)
    codes = _gather_call()(codebook, indices)
    idx_shape = list(x.shape)
    idx_shape[-1] = 1
    return codes.reshape(x.shape), indices.reshape(idx_shape)


# speedup vs baseline: 1.0874x; 1.0011x over previous
import functools

import jax
import jax.numpy as jnp
from jax import lax
from jax.experimental import pallas as pl
from jax.experimental.pallas import tpu as pltpu
from jax.experimental.pallas import tpu_sc as plsc

D = 256
K = 8192
N = 16 * 576

BN = 1024
NN = N // BN
CH = 1024
NCH = K // CH

_INF = float("inf")


def _norms_body(x_ref, a2_ref):
    x = x_ref[...]
    a2_ref[0, :] = jnp.sum(x * x, axis=1)


_norms_call = pl.pallas_call(
    _norms_body,
    out_shape=jax.ShapeDtypeStruct((1, N), jnp.float32),
)


def _preimage_threshold(m2):
    m2c = jnp.maximum(m2, 0.0)
    sk = jnp.sqrt(m2c)
    s_next = lax.bitcast_convert_type(
        lax.bitcast_convert_type(sk, jnp.int32) + 1, jnp.float32)
    pb = lax.bitcast_convert_type(sk * s_next, jnp.int32)
    t = jnp.zeros_like(m2c)
    for db in (-1, 0, 1):
        c = lax.bitcast_convert_type(jnp.maximum(pb + db, 0), jnp.float32)
        t = jnp.where(jnp.sqrt(c) == sk, c, t)
    return jnp.maximum(t, m2c), sk


def _argmin_body(x_ref, cb_ref, a2_ref, out_ref, rowf_s, b2_s):
    n = pl.program_id(0)

    @pl.when(n == 0)
    def _prologue():
        rowf_s[...] = lax.broadcasted_iota(jnp.int32, (CH, BN), 0).astype(
            jnp.float32)
        cb = cb_ref[...]
        b2_s[:, 0] = jnp.sum(cb * cb, axis=1)

    x2 = x_ref[...] * 2.0
    a2 = a2_ref[...]

    best_s = jnp.full((BN,), _INF, dtype=jnp.float32)
    best_i = jnp.zeros((BN,), dtype=jnp.float32)
    for j in range(NCH):
        cbj = cb_ref[pl.ds(j * CH, CH), :]
        s2 = lax.dot_general(cbj, x2, (((1,), (1,)), ((), ())),
                             preferred_element_type=jnp.float32)
        b2j = b2_s[pl.ds(j * CH, CH), :]
        vj = a2 + b2j - s2
        mj = jnp.min(vj, axis=0)
        tj, skj = _preimage_threshold(mj)
        lij = jnp.min(jnp.where(vj <= tj[None, :], rowf_s[...], _INF),
                      axis=0)
        upd = skj < best_s
        best_i = jnp.where(upd, lij + jnp.float32(j * CH), best_i)
        best_s = jnp.where(upd, skj, best_s)

    out_ref[0, 0, :] = best_i.astype(jnp.int32)


_argmin_call = pl.pallas_call(
    _argmin_body,
    grid=(NN,),
    in_specs=[
        pl.BlockSpec((BN, D), lambda n: (n, 0)),
        pl.BlockSpec((K, D), lambda n: (0, 0)),
        pl.BlockSpec((1, BN), lambda n: (0, n)),
    ],
    out_specs=pl.BlockSpec((1, 1, BN), lambda n: (n, 0, 0)),
    out_shape=jax.ShapeDtypeStruct((NN, 1, BN), jnp.int32),
    scratch_shapes=[
        pltpu.VMEM((CH, BN), jnp.float32),
        pltpu.VMEM((K, 1), jnp.float32),
    ],
    compiler_params=pltpu.CompilerParams(
        dimension_semantics=("arbitrary",),
    ),
)


_NW = 32
_BPW = N // _NW


def _gather_body(table_hbm, idx_hbm, out_hbm, idx_v, rows_v, sem):
    wid = lax.axis_index("s") * 2 + lax.axis_index("c")
    base = wid * _BPW
    pltpu.sync_copy(idx_hbm.at[pl.ds(base, _BPW)], idx_v)
    pltpu.async_copy(table_hbm.at[idx_v], rows_v, sem).wait()
    pltpu.sync_copy(rows_v, out_hbm.at[pl.ds(base, _BPW)])


@functools.cache
def _gather_call():
    return functools.partial(
        pl.kernel,
        out_type=jax.ShapeDtypeStruct((N, D), jnp.float32),
        mesh=plsc.VectorSubcoreMesh(core_axis_name="c", subcore_axis_name="s"),
        scratch_types=[
            pltpu.VMEM((_BPW,), jnp.int32),
            pltpu.VMEM((_BPW, D), jnp.float32),
            pltpu.SemaphoreType.DMA,
        ],
    )(_gather_body)


def kernel(x, codebook):
    xf = x.reshape(N, D)
    a2 = _norms_call(xf)
    idx_blocks = _argmin_call(xf, codebook, a2)
    indices = idx_blocks.reshape(N)
    codes = _gather_call()(codebook, indices)
    idx_shape = list(x.shape)
    idx_shape[-1] = 1
    return codes.reshape(x.shape), indices.reshape(idx_shape)
